# Initial kernel scaffold; baseline (speedup 1.0000x reference)
import jax
import jax.numpy as jnp
from jax.experimental import pallas as pl


def kernel(feat, edge_index, W1, b1, W2, b2, fW1, fb1, fW2, fb2):
    raise NotImplementedError("write your pallas kernel here")


# speedup vs baseline: 65.0651x; 65.0651x over previous
import jax
import jax.numpy as jnp
from jax import lax
from jax.experimental import pallas as pl
from jax.experimental.pallas import tpu as pltpu
from jax.experimental.pallas import tpu_sc as plsc

N = 10000
E = 320000
EMB = 128
NUM_CLASSES = 10

NC = 2
NS = 16
L = 16
NW = NC * NS
E_PER = E // NW
E_VECS = E_PER // L
N_VECS = N // L


def _sc_edge_pass(feat_hbm, src_hbm, dst_hbm, agg_out, deg_out,
                  feat_v, src_v, dst_v, agg_v, deg_v):
    wid = lax.axis_index("s") * NC + lax.axis_index("c")
    base = wid * E_PER
    pltpu.sync_copy(feat_hbm, feat_v)
    pltpu.sync_copy(src_hbm.at[pl.ds(base, E_PER)], src_v)
    pltpu.sync_copy(dst_hbm.at[pl.ds(base, E_PER)], dst_v)

    zeros = jnp.zeros((L,), jnp.float32)

    def zero_body(i, carry):
        agg_v[pl.ds(i * L, L)] = zeros
        deg_v[pl.ds(i * L, L)] = zeros
        return carry

    lax.fori_loop(0, N_VECS, zero_body, 0)

    ones = jnp.ones((L,), jnp.float32)

    def body(i, carry):
        si = src_v[pl.ds(i * L, L)]
        di = dst_v[pl.ds(i * L, L)]
        vals = plsc.load_gather(feat_v, [si])
        plsc.addupdate_scatter(agg_v, [di], vals)
        plsc.addupdate_scatter(deg_v, [si], ones)
        return carry

    lax.fori_loop(0, E_VECS, body, 0)

    pltpu.sync_copy(agg_v, agg_out.at[wid])
    pltpu.sync_copy(deg_v, deg_out.at[wid])


_sc_call = pl.kernel(
    _sc_edge_pass,
    out_type=[jax.ShapeDtypeStruct((NW, N), jnp.float32),
              jax.ShapeDtypeStruct((NW, N), jnp.float32)],
    mesh=plsc.VectorSubcoreMesh(core_axis_name="c", subcore_axis_name="s"),
    compiler_params=pltpu.CompilerParams(needs_layout_passes=False),
    scratch_types=[
        pltpu.VMEM((N,), jnp.float32),
        pltpu.VMEM((E_PER,), jnp.int32),
        pltpu.VMEM((E_PER,), jnp.int32),
        pltpu.VMEM((N,), jnp.float32),
        pltpu.VMEM((N,), jnp.float32),
    ],
)


def _tc_head(feat_ref, aggp_ref, degp_ref, W1_ref, b1_ref, W2_ref, b2_ref,
             fW1_ref, fb1_ref, fW2_ref, fb2_ref, out_ref):
    agg = jnp.sum(aggp_ref[...], axis=0)
    deg = jnp.sum(degp_ref[...], axis=0)
    s = feat_ref[...][:, 0] + agg
    w = 1.0 + deg
    h1 = jax.nn.relu(s[:, None] * W1_ref[...] + b1_ref[...])
    t = jnp.dot(w[None, :], h1, preferred_element_type=jnp.float32)
    hg = t * (1.0 / N)
    hg = jnp.dot(hg, W2_ref[...], preferred_element_type=jnp.float32) + b2_ref[...]
    o = jax.nn.relu(
        jnp.dot(hg, fW1_ref[...], preferred_element_type=jnp.float32) + fb1_ref[...])
    out_ref[...] = (
        jnp.dot(o, fW2_ref[...], preferred_element_type=jnp.float32) + fb2_ref[...])


_tc_call = pl.pallas_call(
    _tc_head,
    out_shape=jax.ShapeDtypeStruct((1, NUM_CLASSES), jnp.float32),
)


def kernel(feat, edge_index, W1, b1, W2, b2, fW1, fb1, fW2, fb2):
    src = edge_index[0].astype(jnp.int32)
    dst = edge_index[1].astype(jnp.int32)
    f = feat.astype(jnp.float32)
    aggp, degp = _sc_call(f[:, 0], src, dst)
    return _tc_call(f, aggp, degp,
                    W1, b1.reshape(1, EMB), W2, b2.reshape(1, EMB),
                    fW1, fb1.reshape(1, EMB), fW2, fb2.reshape(1, NUM_CLASSES))


# speedup vs baseline: 74.6202x; 1.1469x over previous
import jax
import jax.numpy as jnp
from jax import lax
from jax.experimental import pallas as pl
from jax.experimental.pallas import tpu as pltpu
from jax.experimental.pallas import tpu_sc as plsc

N = 10000
E = 320000
EMB = 128
NUM_CLASSES = 10

NC = 2
NS = 16
L = 16
NW = NC * NS
E_PER = E // NW
E_VECS = E_PER // L
N_VECS = N // L


UNROLL = 5


def _sc_edge_pass(feat_hbm, ei_hbm, zeros_hbm, agg_out, deg_out,
                  feat_v, src_v, dst_v, agg_v, deg_v):
    wid = lax.axis_index("s") * NC + lax.axis_index("c")
    base = wid * E_PER
    pltpu.sync_copy(feat_hbm, feat_v)
    pltpu.sync_copy(ei_hbm.at[pl.ds(base, E_PER)], src_v)
    pltpu.sync_copy(ei_hbm.at[pl.ds(E + base, E_PER)], dst_v)
    pltpu.sync_copy(zeros_hbm, agg_v)
    pltpu.sync_copy(zeros_hbm, deg_v)

    ones = jnp.ones((L,), jnp.float32)

    def body(i, carry):
        for u in range(UNROLL):
            off = (i * UNROLL + u) * L
            si = src_v[pl.ds(off, L)]
            di = dst_v[pl.ds(off, L)]
            vals = plsc.load_gather(feat_v, [si])
            plsc.addupdate_scatter(agg_v, [di], vals)
            plsc.addupdate_scatter(deg_v, [si], ones)
        return carry

    lax.fori_loop(0, E_VECS // UNROLL, body, 0)

    pltpu.sync_copy(agg_v, agg_out.at[wid])
    pltpu.sync_copy(deg_v, deg_out.at[wid])


_sc_call = pl.kernel(
    _sc_edge_pass,
    out_type=[jax.ShapeDtypeStruct((NW, N), jnp.float32),
              jax.ShapeDtypeStruct((NW, N), jnp.float32)],
    mesh=plsc.VectorSubcoreMesh(core_axis_name="c", subcore_axis_name="s"),
    compiler_params=pltpu.CompilerParams(needs_layout_passes=False),
    scratch_types=[
        pltpu.VMEM((N,), jnp.float32),
        pltpu.VMEM((E_PER,), jnp.int32),
        pltpu.VMEM((E_PER,), jnp.int32),
        pltpu.VMEM((N,), jnp.float32),
        pltpu.VMEM((N,), jnp.float32),
    ],
)


def _tc_head(feat_ref, aggp_ref, degp_ref, W1_ref, b1_ref, W2_ref, b2_ref,
             fW1_ref, fb1_ref, fW2_ref, fb2_ref, out_ref):
    agg = jnp.sum(aggp_ref[...], axis=0)
    deg = jnp.sum(degp_ref[...], axis=0)
    s = feat_ref[...][:, 0] + agg
    w = 1.0 + deg
    h1 = jax.nn.relu(s[:, None] * W1_ref[...] + b1_ref[...])
    t = jnp.dot(w[None, :], h1, preferred_element_type=jnp.float32)
    hg = t * (1.0 / N)
    hg = jnp.dot(hg, W2_ref[...], preferred_element_type=jnp.float32) + b2_ref[...]
    o = jax.nn.relu(
        jnp.dot(hg, fW1_ref[...], preferred_element_type=jnp.float32) + fb1_ref[...])
    out_ref[...] = (
        jnp.dot(o, fW2_ref[...], preferred_element_type=jnp.float32) + fb2_ref[...])


_tc_call = pl.pallas_call(
    _tc_head,
    out_shape=jax.ShapeDtypeStruct((1, NUM_CLASSES), jnp.float32),
)


def kernel(feat, edge_index, W1, b1, W2, b2, fW1, fb1, fW2, fb2):
    ei = edge_index.astype(jnp.int32).reshape(-1)
    f = feat.astype(jnp.float32)
    aggp, degp = _sc_call(f[:, 0], ei, jnp.zeros((N,), jnp.float32))
    return _tc_call(f, aggp, degp,
                    W1, b1.reshape(1, EMB), W2, b2.reshape(1, EMB),
                    fW1, fb1.reshape(1, EMB), fW2, fb2.reshape(1, NUM_CLASSES))
